# Initial kernel scaffold; baseline (speedup 1.0000x reference)
import jax
import jax.numpy as jnp
from jax.experimental import pallas as pl


def kernel(x, edge_index, edge_attr, W1r, b1, W1s, W2r, b2, W2s):
    raise NotImplementedError("write your pallas kernel here")


# speedup vs baseline: 2.9967x; 2.9967x over previous
import functools

import jax
import jax.numpy as jnp
from jax import lax
from jax.experimental import pallas as pl
from jax.experimental.pallas import tpu as pltpu
from jax.experimental.pallas import tpu_sc as plsc

N = 10000
E = 320000
D = 128

NC = 2
NS = 16
NW = NC * NS

K = 128
NCHUNK = 80
EPT = NCHUNK * K
E_PAD = NW * EPT

N_ACC = 10240
ROWS_PER_TILE = N_ACC // NS
ZCHUNK = 128


def _sc_body(x_hbm, src_hbm, dst_hbm, w_hbm, out_hbm,
             src_v, dst_v, w_v, rows_v, acc_sh, sem):
    cid = lax.axis_index("c")
    sid = lax.axis_index("s")
    wid = sid * NC + cid

    pltpu.sync_copy(src_hbm.at[wid], src_v)
    pltpu.sync_copy(dst_hbm.at[wid], dst_v)
    pltpu.sync_copy(w_hbm.at[wid], w_v)

    def _zero_body(i, carry):
        for c in range(D // 16):
            rows_v[i, pl.ds(c * 16, 16)] = jnp.zeros((16,), jnp.float32)
        return carry
    lax.fori_loop(0, ZCHUNK, _zero_body, 0)
    for r in range(ROWS_PER_TILE // ZCHUNK):
        pltpu.sync_copy(rows_v.at[pl.ds(0, ZCHUNK)],
                        acc_sh.at[pl.ds(sid * ROWS_PER_TILE + r * ZCHUNK,
                                        ZCHUNK)])
    plsc.subcore_barrier()

    def _chunk_body(j, carry):
        pltpu.async_copy(x_hbm.at[src_v.at[j, 0]], rows_v, sem).wait()

        def _scale_body(g, c2):
            wv = w_v[j, 0, pl.ds(g * 16, 16)]
            for i in range(16):
                wi = wv[i]
                e = g * 16 + i
                for c in range(D // 16):
                    rows_v[e, pl.ds(c * 16, 16)] = (
                        rows_v[e, pl.ds(c * 16, 16)] * wi)
            return c2
        lax.fori_loop(0, K // 16, _scale_body, 0)

        pltpu.sync_copy(rows_v, acc_sh.at[dst_v.at[j, 0]], add=True)
        return carry
    lax.fori_loop(0, NCHUNK, _chunk_body, 0)
    plsc.subcore_barrier()

    for r in range(ROWS_PER_TILE // ZCHUNK):
        base = sid * ROWS_PER_TILE + r * ZCHUNK
        pltpu.sync_copy(acc_sh.at[pl.ds(base, ZCHUNK)],
                        rows_v.at[pl.ds(0, ZCHUNK)])
        pltpu.sync_copy(rows_v.at[pl.ds(0, ZCHUNK)],
                        out_hbm.at[cid, pl.ds(base, ZCHUNK)])


_sc_segment = functools.partial(
    pl.kernel,
    mesh=plsc.VectorSubcoreMesh(core_axis_name="c", subcore_axis_name="s"),
    out_type=jax.ShapeDtypeStruct((NC, N_ACC, D), jnp.float32),
    scratch_types=[
        pltpu.VMEM((NCHUNK, 1, K), jnp.int32),
        pltpu.VMEM((NCHUNK, 1, K), jnp.int32),
        pltpu.VMEM((NCHUNK, 1, K), jnp.float32),
        pltpu.VMEM((K, D), jnp.float32),
        pltpu.VMEM_SHARED((N_ACC, D), jnp.float32),
        pltpu.SemaphoreType.DMA,
    ],
)(_sc_body)


def _dense_body(p_ref, x_ref, wr_ref, ws_ref, b_ref, o_ref, *, act):
    agg = p_ref[0] + p_ref[1]
    z = jnp.dot(agg, wr_ref[...], preferred_element_type=jnp.float32)
    z = z + jnp.dot(x_ref[...], ws_ref[...], preferred_element_type=jnp.float32)
    z = z + b_ref[...]
    if act:
        z = jnp.where(z > 0, z, jnp.exp(z) - 1.0)
    o_ref[...] = z


def _dense(partials, x, wrT, wsT, b, act):
    R = 1000
    return pl.pallas_call(
        functools.partial(_dense_body, act=act),
        grid=(N // R,),
        in_specs=[
            pl.BlockSpec((NC, R, D), lambda i: (0, i, 0)),
            pl.BlockSpec((R, D), lambda i: (i, 0)),
            pl.BlockSpec((D, D), lambda i: (0, 0)),
            pl.BlockSpec((D, D), lambda i: (0, 0)),
            pl.BlockSpec((1, D), lambda i: (0, 0)),
        ],
        out_specs=pl.BlockSpec((R, D), lambda i: (i, 0)),
        out_shape=jax.ShapeDtypeStruct((N, D), jnp.float32),
    )(partials, x, wrT, wsT, b)


def kernel(x, edge_index, edge_attr, W1r, b1, W1s, W2r, b2, W2s):
    src = edge_index[0].astype(jnp.int32)
    dst = edge_index[1].astype(jnp.int32)
    w = edge_attr.astype(jnp.float32)

    pad = E_PAD - E
    src = jnp.concatenate([src, jnp.zeros((pad,), jnp.int32)])
    dst = jnp.concatenate([dst, jnp.zeros((pad,), jnp.int32)])
    w = jnp.concatenate([w, jnp.zeros((pad,), jnp.float32)])
    srcr = src.reshape(NW, NCHUNK, 1, K)
    dstr = dst.reshape(NW, NCHUNK, 1, K)
    wr = w.reshape(NW, NCHUNK, 1, K)

    w1rT = W1r.T
    w1sT = W1s.T
    w2rT = W2r.T
    w2sT = W2s.T
    b1r = b1.reshape(1, D)
    b2r = b2.reshape(1, D)

    p1 = _sc_segment(x, srcr, dstr, wr)
    h = _dense(p1, x, w1rT, w1sT, b1r, act=True)
    p2 = _sc_segment(h, srcr, dstr, wr)
    out = _dense(p2, h, w2rT, w2sT, b2r, act=False)
    return out


# speedup vs baseline: 3.6671x; 1.2237x over previous
import functools

import jax
import jax.numpy as jnp
from jax import lax
from jax.experimental import pallas as pl
from jax.experimental.pallas import tpu as pltpu
from jax.experimental.pallas import tpu_sc as plsc

N = 10000
E = 320000
D = 128

NC = 2
NS = 16
NW = NC * NS

K = 128
NCHUNK = 80
EPT = NCHUNK * K
E_PAD = NW * EPT

N_ACC = 10240
ROWS_PER_TILE = N_ACC // NS
ZCHUNK = 128

SUPER = 8
NSUPER = NCHUNK // SUPER


def _sc_body(x_hbm, src_hbm, dst_hbm, w_hbm, out_hbm,
             src_v, dst0_v, dst1_v, w0_v, w1_v, rows0_v, rows1_v, acc_sh,
             sem_r0, sem_r1, sem_i0, sem_i1):
    cid = lax.axis_index("c")
    sid = lax.axis_index("s")
    wid = sid * NC + cid

    pltpu.sync_copy(src_hbm.at[wid], src_v)
    pltpu.async_copy(dst_hbm.at[wid, 0], dst0_v, sem_i0)
    pltpu.async_copy(w_hbm.at[wid, 0], w0_v, sem_i0)

    def _zero_body(i, carry):
        for c in range(D // 16):
            rows0_v[i, pl.ds(c * 16, 16)] = jnp.zeros((16,), jnp.float32)
        return carry
    lax.fori_loop(0, ZCHUNK, _zero_body, 0)
    for r in range(ROWS_PER_TILE // ZCHUNK):
        pltpu.sync_copy(rows0_v.at[pl.ds(0, ZCHUNK)],
                        acc_sh.at[pl.ds(sid * ROWS_PER_TILE + r * ZCHUNK,
                                        ZCHUNK)])
    plsc.subcore_barrier()

    pltpu.async_copy(x_hbm.at[src_v.at[0, 0]], rows0_v, sem_r0)
    pltpu.async_copy(x_hbm.at[src_v.at[1, 0]], rows1_v, sem_r1)

    def _scale_scatter(dstb, wb, lp, rows_v):
        def _scale_body(g, c2):
            wv = wb[lp, 0, pl.ds(g * 16, 16)]
            for i in range(16):
                wi = wv[i]
                e = g * 16 + i
                for c in range(D // 16):
                    rows_v[e, pl.ds(c * 16, 16)] = (
                        rows_v[e, pl.ds(c * 16, 16)] * wi)
            return c2
        lax.fori_loop(0, K // 16, _scale_body, 0)
        pltpu.sync_copy(rows_v, acc_sh.at[dstb.at[lp, 0]], add=True)

    def _super(s, dstb, wb, semi, dstb_n, wb_n, semi_n):
        pltpu.make_async_copy(dst_hbm.at[wid, s], dstb, semi).wait()
        pltpu.make_async_copy(w_hbm.at[wid, s], wb, semi).wait()
        sn = lax.rem(s + 1, NSUPER)
        pltpu.async_copy(dst_hbm.at[wid, sn], dstb_n, semi_n)
        pltpu.async_copy(w_hbm.at[wid, sn], wb_n, semi_n)

        def _pair(p2, c):
            lp = 2 * p2
            j = s * SUPER + lp
            pltpu.make_async_copy(
                x_hbm.at[src_v.at[j, 0]], rows0_v, sem_r0).wait()
            _scale_scatter(dstb, wb, lp, rows0_v)
            pltpu.async_copy(
                x_hbm.at[src_v.at[lax.rem(j + 2, NCHUNK), 0]],
                rows0_v, sem_r0)
            pltpu.make_async_copy(
                x_hbm.at[src_v.at[j + 1, 0]], rows1_v, sem_r1).wait()
            _scale_scatter(dstb, wb, lp + 1, rows1_v)
            pltpu.async_copy(
                x_hbm.at[src_v.at[lax.rem(j + 3, NCHUNK), 0]],
                rows1_v, sem_r1)
            return c
        lax.fori_loop(0, SUPER // 2, _pair, 0)

    def _souter(s2, c):
        s = 2 * s2
        _super(s, dst0_v, w0_v, sem_i0, dst1_v, w1_v, sem_i1)
        _super(s + 1, dst1_v, w1_v, sem_i1, dst0_v, w0_v, sem_i0)
        return c
    lax.fori_loop(0, NSUPER // 2, _souter, 0)

    pltpu.make_async_copy(x_hbm.at[src_v.at[0, 0]], rows0_v, sem_r0).wait()
    pltpu.make_async_copy(x_hbm.at[src_v.at[1, 0]], rows1_v, sem_r1).wait()
    pltpu.make_async_copy(dst_hbm.at[wid, 0], dst0_v, sem_i0).wait()
    pltpu.make_async_copy(w_hbm.at[wid, 0], w0_v, sem_i0).wait()
    plsc.subcore_barrier()

    for r in range(ROWS_PER_TILE // ZCHUNK):
        base = sid * ROWS_PER_TILE + r * ZCHUNK
        pltpu.sync_copy(acc_sh.at[pl.ds(base, ZCHUNK)],
                        rows0_v.at[pl.ds(0, ZCHUNK)])
        pltpu.sync_copy(rows0_v.at[pl.ds(0, ZCHUNK)],
                        out_hbm.at[cid, pl.ds(base, ZCHUNK)])


_sc_segment = functools.partial(
    pl.kernel,
    mesh=plsc.VectorSubcoreMesh(core_axis_name="c", subcore_axis_name="s"),
    out_type=jax.ShapeDtypeStruct((NC, N_ACC, D), jnp.float32),
    scratch_types=[
        pltpu.VMEM((NCHUNK, 1, K), jnp.int32),
        pltpu.VMEM((SUPER, 1, K), jnp.int32),
        pltpu.VMEM((SUPER, 1, K), jnp.int32),
        pltpu.VMEM((SUPER, 1, K), jnp.float32),
        pltpu.VMEM((SUPER, 1, K), jnp.float32),
        pltpu.VMEM((K, D), jnp.float32),
        pltpu.VMEM((K, D), jnp.float32),
        pltpu.VMEM_SHARED((N_ACC, D), jnp.float32),
        pltpu.SemaphoreType.DMA,
        pltpu.SemaphoreType.DMA,
        pltpu.SemaphoreType.DMA,
        pltpu.SemaphoreType.DMA,
    ],
)(_sc_body)


def _dense_body(p_ref, x_ref, wr_ref, ws_ref, b_ref, o_ref, *, act):
    agg = p_ref[0] + p_ref[1]
    z = jnp.dot(agg, wr_ref[...], preferred_element_type=jnp.float32)
    z = z + jnp.dot(x_ref[...], ws_ref[...], preferred_element_type=jnp.float32)
    z = z + b_ref[...]
    if act:
        z = jnp.where(z > 0, z, jnp.exp(z) - 1.0)
    o_ref[...] = z


def _dense(partials, x, wrT, wsT, b, act):
    R = 1000
    return pl.pallas_call(
        functools.partial(_dense_body, act=act),
        grid=(N // R,),
        in_specs=[
            pl.BlockSpec((NC, R, D), lambda i: (0, i, 0)),
            pl.BlockSpec((R, D), lambda i: (i, 0)),
            pl.BlockSpec((D, D), lambda i: (0, 0)),
            pl.BlockSpec((D, D), lambda i: (0, 0)),
            pl.BlockSpec((1, D), lambda i: (0, 0)),
        ],
        out_specs=pl.BlockSpec((R, D), lambda i: (i, 0)),
        out_shape=jax.ShapeDtypeStruct((N, D), jnp.float32),
    )(partials, x, wrT, wsT, b)


def kernel(x, edge_index, edge_attr, W1r, b1, W1s, W2r, b2, W2s):
    src = edge_index[0].astype(jnp.int32)
    dst = edge_index[1].astype(jnp.int32)
    w = edge_attr.astype(jnp.float32)

    pad = E_PAD - E
    src = jnp.concatenate([src, jnp.zeros((pad,), jnp.int32)])
    dst = jnp.concatenate([dst, jnp.zeros((pad,), jnp.int32)])
    w = jnp.concatenate([w, jnp.zeros((pad,), jnp.float32)])
    srcr = src.reshape(NW, NCHUNK, 1, K)
    dstr = dst.reshape(NW, NSUPER, SUPER, 1, K)
    wr = w.reshape(NW, NSUPER, SUPER, 1, K)

    w1rT = W1r.T
    w1sT = W1s.T
    w2rT = W2r.T
    w2sT = W2s.T
    b1r = b1.reshape(1, D)
    b2r = b2.reshape(1, D)

    p1 = _sc_segment(x, srcr, dstr, wr)
    h = _dense(p1, x, w1rT, w1sT, b1r, act=True)
    p2 = _sc_segment(h, srcr, dstr, wr)
    out = _dense(p2, h, w2rT, w2sT, b2r, act=False)
    return out


# speedup vs baseline: 3.6702x; 1.0008x over previous
import functools

import jax
import jax.numpy as jnp
from jax import lax
from jax.experimental import pallas as pl
from jax.experimental.pallas import tpu as pltpu
from jax.experimental.pallas import tpu_sc as plsc

N = 10000
E = 320000
D = 128

NC = 2
NS = 16
NW = NC * NS

K = 128
NCHUNK = 80
EPT = NCHUNK * K
E_PAD = NW * EPT

N_ACC = 10240
ROWS_PER_TILE = N_ACC // NS
ZCHUNK = 128

SUPER = 8
NSUPER = NCHUNK // SUPER


def _sc_body(x_hbm, src_hbm, dst_hbm, w_hbm, out_hbm,
             src_v, dst0_v, dst1_v, w0_v, w1_v, rows0_v, rows1_v, acc_sh,
             sem_r0, sem_r1, sem_i0, sem_i1):
    cid = lax.axis_index("c")
    sid = lax.axis_index("s")
    wid = sid * NC + cid

    with jax.named_scope("idx_slab"):
        pltpu.sync_copy(src_hbm.at[wid], src_v)
        pltpu.async_copy(dst_hbm.at[wid, 0], dst0_v, sem_i0)
        pltpu.async_copy(w_hbm.at[wid, 0], w0_v, sem_i0)

    with jax.named_scope("zero_acc"):
        def _zero_body(i, carry):
            for c in range(D // 16):
                rows0_v[i, pl.ds(c * 16, 16)] = jnp.zeros((16,), jnp.float32)
            return carry
        lax.fori_loop(0, ZCHUNK, _zero_body, 0)
        for r in range(ROWS_PER_TILE // ZCHUNK):
            pltpu.sync_copy(rows0_v.at[pl.ds(0, ZCHUNK)],
                            acc_sh.at[pl.ds(sid * ROWS_PER_TILE + r * ZCHUNK,
                                            ZCHUNK)])
        plsc.subcore_barrier()

    pltpu.async_copy(x_hbm.at[src_v.at[0, 0]], rows0_v, sem_r0)
    pltpu.async_copy(x_hbm.at[src_v.at[1, 0]], rows1_v, sem_r1)

    def _scale_scatter(dstb, wb, lp, rows_v):
        def _scale_body(g, c2):
            wv = wb[lp, 0, pl.ds(g * 16, 16)]
            for i in range(16):
                wi = wv[i]
                e = g * 16 + i
                for c in range(D // 16):
                    rows_v[e, pl.ds(c * 16, 16)] = (
                        rows_v[e, pl.ds(c * 16, 16)] * wi)
            return c2
        lax.fori_loop(0, K // 16, _scale_body, 0)
        pltpu.sync_copy(rows_v, acc_sh.at[dstb.at[lp, 0]], add=True)

    def _super(s, dstb, wb, semi, dstb_n, wb_n, semi_n):
        pltpu.make_async_copy(dst_hbm.at[wid, s], dstb, semi).wait()
        pltpu.make_async_copy(w_hbm.at[wid, s], wb, semi).wait()
        sn = lax.rem(s + 1, NSUPER)
        pltpu.async_copy(dst_hbm.at[wid, sn], dstb_n, semi_n)
        pltpu.async_copy(w_hbm.at[wid, sn], wb_n, semi_n)

        def _pair(p2, c):
            lp = 2 * p2
            j = s * SUPER + lp
            pltpu.make_async_copy(
                x_hbm.at[src_v.at[j, 0]], rows0_v, sem_r0).wait()
            _scale_scatter(dstb, wb, lp, rows0_v)
            pltpu.async_copy(
                x_hbm.at[src_v.at[lax.rem(j + 2, NCHUNK), 0]],
                rows0_v, sem_r0)
            pltpu.make_async_copy(
                x_hbm.at[src_v.at[j + 1, 0]], rows1_v, sem_r1).wait()
            _scale_scatter(dstb, wb, lp + 1, rows1_v)
            pltpu.async_copy(
                x_hbm.at[src_v.at[lax.rem(j + 3, NCHUNK), 0]],
                rows1_v, sem_r1)
            return c
        lax.fori_loop(0, SUPER // 2, _pair, 0)

    def _souter(s2, c):
        s = 2 * s2
        _super(s, dst0_v, w0_v, sem_i0, dst1_v, w1_v, sem_i1)
        _super(s + 1, dst1_v, w1_v, sem_i1, dst0_v, w0_v, sem_i0)
        return c
    with jax.named_scope("mainloop"):
        lax.fori_loop(0, NSUPER // 2, _souter, 0)

    pltpu.make_async_copy(x_hbm.at[src_v.at[0, 0]], rows0_v, sem_r0).wait()
    pltpu.make_async_copy(x_hbm.at[src_v.at[1, 0]], rows1_v, sem_r1).wait()
    pltpu.make_async_copy(dst_hbm.at[wid, 0], dst0_v, sem_i0).wait()
    pltpu.make_async_copy(w_hbm.at[wid, 0], w0_v, sem_i0).wait()
    plsc.subcore_barrier()

    with jax.named_scope("copyout"):
     for r in range(ROWS_PER_TILE // ZCHUNK):
        base = sid * ROWS_PER_TILE + r * ZCHUNK
        pltpu.sync_copy(acc_sh.at[pl.ds(base, ZCHUNK)],
                        rows0_v.at[pl.ds(0, ZCHUNK)])
        pltpu.sync_copy(rows0_v.at[pl.ds(0, ZCHUNK)],
                        out_hbm.at[cid, pl.ds(base, ZCHUNK)])


_sc_segment = functools.partial(
    pl.kernel,
    mesh=plsc.VectorSubcoreMesh(core_axis_name="c", subcore_axis_name="s"),
    out_type=jax.ShapeDtypeStruct((NC, N_ACC, D), jnp.float32),
    scratch_types=[
        pltpu.VMEM((NCHUNK, 1, K), jnp.int32),
        pltpu.VMEM((SUPER, 1, K), jnp.int32),
        pltpu.VMEM((SUPER, 1, K), jnp.int32),
        pltpu.VMEM((SUPER, 1, K), jnp.float32),
        pltpu.VMEM((SUPER, 1, K), jnp.float32),
        pltpu.VMEM((K, D), jnp.float32),
        pltpu.VMEM((K, D), jnp.float32),
        pltpu.VMEM_SHARED((N_ACC, D), jnp.float32),
        pltpu.SemaphoreType.DMA,
        pltpu.SemaphoreType.DMA,
        pltpu.SemaphoreType.DMA,
        pltpu.SemaphoreType.DMA,
    ],
)(_sc_body)


def _dense_body(p_ref, x_ref, wr_ref, ws_ref, b_ref, o_ref, *, act):
    agg = p_ref[0] + p_ref[1]
    z = jnp.dot(agg, wr_ref[...], preferred_element_type=jnp.float32)
    z = z + jnp.dot(x_ref[...], ws_ref[...], preferred_element_type=jnp.float32)
    z = z + b_ref[...]
    if act:
        z = jnp.where(z > 0, z, jnp.exp(z) - 1.0)
    o_ref[...] = z


def _dense(partials, x, wrT, wsT, b, act):
    R = 1000
    return pl.pallas_call(
        functools.partial(_dense_body, act=act),
        grid=(N // R,),
        in_specs=[
            pl.BlockSpec((NC, R, D), lambda i: (0, i, 0)),
            pl.BlockSpec((R, D), lambda i: (i, 0)),
            pl.BlockSpec((D, D), lambda i: (0, 0)),
            pl.BlockSpec((D, D), lambda i: (0, 0)),
            pl.BlockSpec((1, D), lambda i: (0, 0)),
        ],
        out_specs=pl.BlockSpec((R, D), lambda i: (i, 0)),
        out_shape=jax.ShapeDtypeStruct((N, D), jnp.float32),
    )(partials, x, wrT, wsT, b)


def kernel(x, edge_index, edge_attr, W1r, b1, W1s, W2r, b2, W2s):
    src = edge_index[0].astype(jnp.int32)
    dst = edge_index[1].astype(jnp.int32)
    w = edge_attr.astype(jnp.float32)

    pad = E_PAD - E
    src = jnp.concatenate([src, jnp.zeros((pad,), jnp.int32)])
    dst = jnp.concatenate([dst, jnp.zeros((pad,), jnp.int32)])
    w = jnp.concatenate([w, jnp.zeros((pad,), jnp.float32)])
    srcr = src.reshape(NW, NCHUNK, 1, K)
    dstr = dst.reshape(NW, NSUPER, SUPER, 1, K)
    wr = w.reshape(NW, NSUPER, SUPER, 1, K)

    w1rT = W1r.T
    w1sT = W1s.T
    w2rT = W2r.T
    w2sT = W2s.T
    b1r = b1.reshape(1, D)
    b2r = b2.reshape(1, D)

    p1 = _sc_segment(x, srcr, dstr, wr)
    h = _dense(p1, x, w1rT, w1sT, b1r, act=True)
    p2 = _sc_segment(h, srcr, dstr, wr)
    out = _dense(p2, h, w2rT, w2sT, b2r, act=False)
    return out


# speedup vs baseline: 3.6865x; 1.0045x over previous
import functools

import jax
import jax.numpy as jnp
from jax import lax
from jax.experimental import pallas as pl
from jax.experimental.pallas import tpu as pltpu
from jax.experimental.pallas import tpu_sc as plsc

N = 10000
E = 320000
D = 128

NC = 2
NS = 16
NW = NC * NS

K = 128
NCHUNK = 80
EPT = NCHUNK * K
E_PAD = NW * EPT

N_ACC = 10240
ROWS_PER_TILE = N_ACC // NS
ZCHUNK = 128

SUPER = 8
NSUPER = NCHUNK // SUPER


def _sc_body(x_hbm, src_hbm, dst_hbm, w_hbm, out_hbm,
             src_v, dst0_v, dst1_v, w0_v, w1_v, rows0_v, rows1_v, acc_sh,
             sem_r0, sem_r1, sem_i0, sem_i1):
    cid = lax.axis_index("c")
    sid = lax.axis_index("s")
    wid = sid * NC + cid

    with jax.named_scope("idx_slab"):
        pltpu.sync_copy(src_hbm.at[wid], src_v)
        pltpu.async_copy(dst_hbm.at[wid, 0], dst0_v, sem_i0)
        pltpu.async_copy(w_hbm.at[wid, 0], w0_v, sem_i0)

    with jax.named_scope("zero_acc"):
        def _zero_body(i, carry):
            for c in range(D // 16):
                rows0_v[i, pl.ds(c * 16, 16)] = jnp.zeros((16,), jnp.float32)
            return carry
        lax.fori_loop(0, ZCHUNK, _zero_body, 0)
        for r in range(ROWS_PER_TILE // ZCHUNK):
            pltpu.sync_copy(rows0_v.at[pl.ds(0, ZCHUNK)],
                            acc_sh.at[pl.ds(sid * ROWS_PER_TILE + r * ZCHUNK,
                                            ZCHUNK)])
        plsc.subcore_barrier()

    pltpu.async_copy(x_hbm.at[src_v.at[0, 0]], rows0_v, sem_r0)
    pltpu.async_copy(x_hbm.at[src_v.at[1, 0]], rows1_v, sem_r1)

    def _scale_scatter(dstb, wb, lp, rows_v):
        def _scale_body(g, c2):
            wv = wb[lp, 0, pl.ds(g * 16, 16)]
            for i in range(16):
                wi = wv[i]
                e = g * 16 + i
                for c in range(D // 16):
                    rows_v[e, pl.ds(c * 16, 16)] = (
                        rows_v[e, pl.ds(c * 16, 16)] * wi)
            return c2
        lax.fori_loop(0, K // 16, _scale_body, 0)
        pltpu.sync_copy(rows_v, acc_sh.at[dstb.at[lp, 0]], add=True)

    def _super(s, dstb, wb, semi, dstb_n, wb_n, semi_n):
        pltpu.make_async_copy(dst_hbm.at[wid, s], dstb, semi).wait()
        pltpu.make_async_copy(w_hbm.at[wid, s], wb, semi).wait()
        sn = s + 1

        @pl.when(sn < NSUPER)
        def _prefetch_idx():
            pltpu.async_copy(dst_hbm.at[wid, sn], dstb_n, semi_n)
            pltpu.async_copy(w_hbm.at[wid, sn], wb_n, semi_n)

        def _pair(p2, c):
            lp = 2 * p2
            j = s * SUPER + lp
            pltpu.make_async_copy(
                x_hbm.at[src_v.at[j, 0]], rows0_v, sem_r0).wait()
            _scale_scatter(dstb, wb, lp, rows0_v)

            @pl.when(j + 2 < NCHUNK)
            def _pf0():
                pltpu.async_copy(
                    x_hbm.at[src_v.at[j + 2, 0]], rows0_v, sem_r0)
            pltpu.make_async_copy(
                x_hbm.at[src_v.at[j + 1, 0]], rows1_v, sem_r1).wait()
            _scale_scatter(dstb, wb, lp + 1, rows1_v)

            @pl.when(j + 3 < NCHUNK)
            def _pf1():
                pltpu.async_copy(
                    x_hbm.at[src_v.at[j + 3, 0]], rows1_v, sem_r1)
            return c
        lax.fori_loop(0, SUPER // 2, _pair, 0)

    def _souter(s2, c):
        s = 2 * s2
        _super(s, dst0_v, w0_v, sem_i0, dst1_v, w1_v, sem_i1)
        _super(s + 1, dst1_v, w1_v, sem_i1, dst0_v, w0_v, sem_i0)
        return c
    with jax.named_scope("mainloop"):
        lax.fori_loop(0, NSUPER // 2, _souter, 0)

    plsc.subcore_barrier()

    with jax.named_scope("copyout"):
     for r in range(ROWS_PER_TILE // ZCHUNK):
        base = sid * ROWS_PER_TILE + r * ZCHUNK
        pltpu.sync_copy(acc_sh.at[pl.ds(base, ZCHUNK)],
                        rows0_v.at[pl.ds(0, ZCHUNK)])
        pltpu.sync_copy(rows0_v.at[pl.ds(0, ZCHUNK)],
                        out_hbm.at[cid, pl.ds(base, ZCHUNK)])


_sc_segment = functools.partial(
    pl.kernel,
    mesh=plsc.VectorSubcoreMesh(core_axis_name="c", subcore_axis_name="s"),
    out_type=jax.ShapeDtypeStruct((NC, N_ACC, D), jnp.float32),
    scratch_types=[
        pltpu.VMEM((NCHUNK, 1, K), jnp.int32),
        pltpu.VMEM((SUPER, 1, K), jnp.int32),
        pltpu.VMEM((SUPER, 1, K), jnp.int32),
        pltpu.VMEM((SUPER, 1, K), jnp.float32),
        pltpu.VMEM((SUPER, 1, K), jnp.float32),
        pltpu.VMEM((K, D), jnp.float32),
        pltpu.VMEM((K, D), jnp.float32),
        pltpu.VMEM_SHARED((N_ACC, D), jnp.float32),
        pltpu.SemaphoreType.DMA,
        pltpu.SemaphoreType.DMA,
        pltpu.SemaphoreType.DMA,
        pltpu.SemaphoreType.DMA,
    ],
)(_sc_body)


def _dense_body(p_ref, x_ref, wr_ref, ws_ref, b_ref, o_ref, *, act):
    agg = p_ref[0] + p_ref[1]
    z = jnp.dot(agg, wr_ref[...], preferred_element_type=jnp.float32)
    z = z + jnp.dot(x_ref[...], ws_ref[...], preferred_element_type=jnp.float32)
    z = z + b_ref[...]
    if act:
        z = jnp.where(z > 0, z, jnp.exp(z) - 1.0)
    o_ref[...] = z


def _dense(partials, x, wrT, wsT, b, act):
    R = 1000
    return pl.pallas_call(
        functools.partial(_dense_body, act=act),
        grid=(N // R,),
        in_specs=[
            pl.BlockSpec((NC, R, D), lambda i: (0, i, 0)),
            pl.BlockSpec((R, D), lambda i: (i, 0)),
            pl.BlockSpec((D, D), lambda i: (0, 0)),
            pl.BlockSpec((D, D), lambda i: (0, 0)),
            pl.BlockSpec((1, D), lambda i: (0, 0)),
        ],
        out_specs=pl.BlockSpec((R, D), lambda i: (i, 0)),
        out_shape=jax.ShapeDtypeStruct((N, D), jnp.float32),
    )(partials, x, wrT, wsT, b)


def kernel(x, edge_index, edge_attr, W1r, b1, W1s, W2r, b2, W2s):
    src = edge_index[0].astype(jnp.int32)
    dst = edge_index[1].astype(jnp.int32)
    w = edge_attr.astype(jnp.float32)

    pad = E_PAD - E
    src = jnp.concatenate([src, jnp.zeros((pad,), jnp.int32)])
    dst = jnp.concatenate([dst, jnp.zeros((pad,), jnp.int32)])
    w = jnp.concatenate([w, jnp.zeros((pad,), jnp.float32)])
    srcr = src.reshape(NW, NCHUNK, 1, K)
    dstr = dst.reshape(NW, NSUPER, SUPER, 1, K)
    wr = w.reshape(NW, NSUPER, SUPER, 1, K)

    w1rT = W1r.T
    w1sT = W1s.T
    w2rT = W2r.T
    w2sT = W2s.T
    b1r = b1.reshape(1, D)
    b2r = b2.reshape(1, D)

    p1 = _sc_segment(x, srcr, dstr, wr)
    h = _dense(p1, x, w1rT, w1sT, b1r, act=True)
    p2 = _sc_segment(h, srcr, dstr, wr)
    out = _dense(p2, h, w2rT, w2sT, b2r, act=False)
    return out


# speedup vs baseline: 10.5203x; 2.8537x over previous
import functools

import jax
import jax.numpy as jnp
from jax import lax
from jax.experimental import pallas as pl
from jax.experimental.pallas import tpu as pltpu
from jax.experimental.pallas import tpu_sc as plsc

N = 10000
E = 320000
D = 128

NC = 2
NS = 16
NW = NC * NS

K = 128
NCHUNK = 80
EPT = NCHUNK * K
E_PAD = NW * EPT

N_ACC = 10240
ROWS_PER_TILE = N_ACC // NS
ZCHUNK = 128

SUPER = 8
NSUPER = NCHUNK // SUPER


def _sc_body(x_hbm, src_hbm, dst_hbm, w_hbm, out_hbm,
             src_v, dst0_v, dst1_v, w0_v, w1_v, rows0_v, rows1_v, acc_sh,
             sem_r0, sem_r1, sem_i0, sem_i1):
    cid = lax.axis_index("c")
    sid = lax.axis_index("s")
    wid = sid * NC + cid

    with jax.named_scope("idx_slab"):
        pltpu.sync_copy(src_hbm.at[wid], src_v)
        pltpu.async_copy(dst_hbm.at[wid, 0], dst0_v, sem_i0)
        pltpu.async_copy(w_hbm.at[wid, 0], w0_v, sem_i0)

    with jax.named_scope("zero_acc"):
        def _zero_body(i, carry):
            for c in range(D // 16):
                rows0_v[i, pl.ds(c * 16, 16)] = jnp.zeros((16,), jnp.float32)
            return carry
        lax.fori_loop(0, ZCHUNK, _zero_body, 0)
        for r in range(ROWS_PER_TILE // ZCHUNK):
            pltpu.sync_copy(rows0_v.at[pl.ds(0, ZCHUNK)],
                            acc_sh.at[pl.ds(sid * ROWS_PER_TILE + r * ZCHUNK,
                                            ZCHUNK)])
        plsc.subcore_barrier()

    pltpu.async_copy(x_hbm.at[src_v.at[0, 0]], rows0_v, sem_r0)
    pltpu.async_copy(x_hbm.at[src_v.at[1, 0]], rows1_v, sem_r1)

    def _scale_scatter(dstb, wb, lp, rows_v):
        def _scale_body(g, c2):
            wv = wb[lp, 0, pl.ds(g * 16, 16)]
            for i in range(16):
                wi = wv[i]
                e = g * 16 + i
                for c in range(D // 16):
                    rows_v[e, pl.ds(c * 16, 16)] = (
                        rows_v[e, pl.ds(c * 16, 16)] * wi)
            return c2
        lax.fori_loop(0, K // 16, _scale_body, 0)
        pltpu.sync_copy(rows_v, acc_sh.at[dstb.at[lp, 0]], add=True)

    def _super(s, dstb, wb, semi, dstb_n, wb_n, semi_n):
        pltpu.make_async_copy(dst_hbm.at[wid, s], dstb, semi).wait()
        pltpu.make_async_copy(w_hbm.at[wid, s], wb, semi).wait()
        sn = s + 1

        @pl.when(sn < NSUPER)
        def _prefetch_idx():
            pltpu.async_copy(dst_hbm.at[wid, sn], dstb_n, semi_n)
            pltpu.async_copy(w_hbm.at[wid, sn], wb_n, semi_n)

        def _pair(p2, c):
            lp = 2 * p2
            j = s * SUPER + lp
            pltpu.make_async_copy(
                x_hbm.at[src_v.at[j, 0]], rows0_v, sem_r0).wait()
            _scale_scatter(dstb, wb, lp, rows0_v)

            @pl.when(j + 2 < NCHUNK)
            def _pf0():
                pltpu.async_copy(
                    x_hbm.at[src_v.at[j + 2, 0]], rows0_v, sem_r0)
            pltpu.make_async_copy(
                x_hbm.at[src_v.at[j + 1, 0]], rows1_v, sem_r1).wait()
            _scale_scatter(dstb, wb, lp + 1, rows1_v)

            @pl.when(j + 3 < NCHUNK)
            def _pf1():
                pltpu.async_copy(
                    x_hbm.at[src_v.at[j + 3, 0]], rows1_v, sem_r1)
            return c
        lax.fori_loop(0, SUPER // 2, _pair, 0)

    def _souter(s2, c):
        s = 2 * s2
        _super(s, dst0_v, w0_v, sem_i0, dst1_v, w1_v, sem_i1)
        _super(s + 1, dst1_v, w1_v, sem_i1, dst0_v, w0_v, sem_i0)
        return c
    with jax.named_scope("mainloop"):
        lax.fori_loop(0, NSUPER // 2, _souter, 0)

    plsc.subcore_barrier()

    with jax.named_scope("copyout"):
     for r in range(ROWS_PER_TILE // ZCHUNK):
        base = sid * ROWS_PER_TILE + r * ZCHUNK
        pltpu.sync_copy(acc_sh.at[pl.ds(base, ZCHUNK)],
                        rows0_v.at[pl.ds(0, ZCHUNK)])
        pltpu.sync_copy(rows0_v.at[pl.ds(0, ZCHUNK)],
                        out_hbm.at[cid, pl.ds(base, ZCHUNK)])


_sc_segment = functools.partial(
    pl.kernel,
    mesh=plsc.VectorSubcoreMesh(core_axis_name="c", subcore_axis_name="s"),
    out_type=jax.ShapeDtypeStruct((NC, N_ACC, D), jnp.float32),
    scratch_types=[
        pltpu.VMEM((NCHUNK, 1, K), jnp.int32),
        pltpu.VMEM((SUPER, 1, K), jnp.int32),
        pltpu.VMEM((SUPER, 1, K), jnp.int32),
        pltpu.VMEM((SUPER, 1, K), jnp.float32),
        pltpu.VMEM((SUPER, 1, K), jnp.float32),
        pltpu.VMEM((K, D), jnp.float32),
        pltpu.VMEM((K, D), jnp.float32),
        pltpu.VMEM_SHARED((N_ACC, D), jnp.float32),
        pltpu.SemaphoreType.DMA,
        pltpu.SemaphoreType.DMA,
        pltpu.SemaphoreType.DMA,
        pltpu.SemaphoreType.DMA,
    ],
)(_sc_body)


def _dense_body(p_ref, x_ref, wr_ref, ws_ref, b_ref, o_ref, *, act):
    agg = p_ref[0] + p_ref[1]
    z = jnp.dot(agg, wr_ref[...], preferred_element_type=jnp.float32)
    z = z + jnp.dot(x_ref[...], ws_ref[...], preferred_element_type=jnp.float32)
    z = z + b_ref[...]
    if act:
        z = jnp.where(z > 0, z, jnp.exp(z) - 1.0)
    o_ref[...] = z


def _dense(partials, x, wrT, wsT, b, act):
    R = 1000
    return pl.pallas_call(
        functools.partial(_dense_body, act=act),
        grid=(N // R,),
        in_specs=[
            pl.BlockSpec((NC, R, D), lambda i: (0, i, 0)),
            pl.BlockSpec((R, D), lambda i: (i, 0)),
            pl.BlockSpec((D, D), lambda i: (0, 0)),
            pl.BlockSpec((D, D), lambda i: (0, 0)),
            pl.BlockSpec((1, D), lambda i: (0, 0)),
        ],
        out_specs=pl.BlockSpec((R, D), lambda i: (i, 0)),
        out_shape=jax.ShapeDtypeStruct((N, D), jnp.float32),
    )(partials, x, wrT, wsT, b)


def kernel(x, edge_index, edge_attr, W1r, b1, W1s, W2r, b2, W2s):
    src = edge_index[0].astype(jnp.int32)
    dst = edge_index[1].astype(jnp.int32)
    w = edge_attr.astype(jnp.float32)

    pad = E_PAD - E
    spread = jnp.arange(pad, dtype=jnp.int32) % N
    src = jnp.concatenate([src, spread])
    dst = jnp.concatenate([dst, spread])
    w = jnp.concatenate([w, jnp.zeros((pad,), jnp.float32)])
    srcr = src.reshape(NW, NCHUNK, 1, K)
    dstr = dst.reshape(NW, NSUPER, SUPER, 1, K)
    wr = w.reshape(NW, NSUPER, SUPER, 1, K)

    w1rT = W1r.T
    w1sT = W1s.T
    w2rT = W2r.T
    w2sT = W2s.T
    b1r = b1.reshape(1, D)
    b2r = b2.reshape(1, D)

    p1 = _sc_segment(x, srcr, dstr, wr)
    h = _dense(p1, x, w1rT, w1sT, b1r, act=True)
    p2 = _sc_segment(h, srcr, dstr, wr)
    out = _dense(p2, h, w2rT, w2sT, b2r, act=False)
    return out
